# Initial kernel scaffold; baseline (speedup 1.0000x reference)
import functools

import jax
import jax.numpy as jnp
from jax import lax
from jax.experimental import pallas as pl
from jax.experimental.pallas import tpu as pltpu
from jax.experimental.pallas import tpu_sc as plsc

N = 10000
E = 320000
H = 128
NC = 2
NS = 16
NW = NC * NS
CH = 80
NCHUNK = E // (NW * CH)
RPT = N // NS
BLK = 1000
GRID = N // BLK

_MESH = plsc.VectorSubcoreMesh(
    core_axis_name="c", subcore_axis_name="s", num_cores=NC, num_subcores=NS
)


def _sc_agg_deg_body(h_hbm, src_hbm, dst_hbm, zeros_hbm, zdeg_hbm, ones_hbm,
                     out_agg, out_deg,
                     src_v, dst_v, rows_v, ones_v, acc, accd, sem):
    c = lax.axis_index("c")
    s = lax.axis_index("s")
    w = c * NS + s
    r0 = s * RPT
    pltpu.sync_copy(zeros_hbm.at[pl.ds(r0, RPT), :], acc.at[pl.ds(r0, RPT), :])
    pltpu.sync_copy(zdeg_hbm.at[pl.ds(r0, RPT), :], accd.at[pl.ds(r0, RPT), :])
    pltpu.sync_copy(src_hbm.at[w], src_v)
    pltpu.sync_copy(dst_hbm.at[w], dst_v)
    pltpu.sync_copy(ones_hbm, ones_v)
    plsc.subcore_barrier()

    def step(j, carry):
        pltpu.async_copy(h_hbm.at[src_v.at[j]], rows_v, sem).wait()
        pltpu.sync_copy(rows_v, acc.at[dst_v.at[j]], add=True)
        pltpu.sync_copy(ones_v, accd.at[dst_v.at[j]], add=True)
        return carry

    lax.fori_loop(0, NCHUNK, step, 0)
    plsc.subcore_barrier()
    pltpu.sync_copy(acc.at[pl.ds(r0, RPT), :], out_agg.at[c, pl.ds(r0, RPT), :])
    pltpu.sync_copy(accd.at[pl.ds(r0, RPT), :], out_deg.at[c, pl.ds(r0, RPT), :])


def _sc_agg_body(h_hbm, src_hbm, dst_hbm, zeros_hbm,
                 out_agg,
                 src_v, dst_v, rows_v, acc, sem):
    c = lax.axis_index("c")
    s = lax.axis_index("s")
    w = c * NS + s
    r0 = s * RPT
    pltpu.sync_copy(zeros_hbm.at[pl.ds(r0, RPT), :], acc.at[pl.ds(r0, RPT), :])
    pltpu.sync_copy(src_hbm.at[w], src_v)
    pltpu.sync_copy(dst_hbm.at[w], dst_v)
    plsc.subcore_barrier()

    def step(j, carry):
        pltpu.async_copy(h_hbm.at[src_v.at[j]], rows_v, sem).wait()
        pltpu.sync_copy(rows_v, acc.at[dst_v.at[j]], add=True)
        return carry

    lax.fori_loop(0, NCHUNK, step, 0)
    plsc.subcore_barrier()
    pltpu.sync_copy(acc.at[pl.ds(r0, RPT), :], out_agg.at[c, pl.ds(r0, RPT), :])


_sc_agg_deg = pl.kernel(
    _sc_agg_deg_body,
    out_type=(
        jax.ShapeDtypeStruct((NC, N, H), jnp.float32),
        jax.ShapeDtypeStruct((NC, N, 16), jnp.float32),
    ),
    mesh=_MESH,
    scratch_types=[
        pltpu.VMEM((NCHUNK, CH), jnp.int32),
        pltpu.VMEM((NCHUNK, CH), jnp.int32),
        pltpu.VMEM((CH, H), jnp.float32),
        pltpu.VMEM((CH, 16), jnp.float32),
        pltpu.VMEM_SHARED((N, H), jnp.float32),
        pltpu.VMEM_SHARED((N, 16), jnp.float32),
        pltpu.SemaphoreType.DMA,
    ],
)

_sc_agg = pl.kernel(
    _sc_agg_body,
    out_type=jax.ShapeDtypeStruct((NC, N, H), jnp.float32),
    mesh=_MESH,
    scratch_types=[
        pltpu.VMEM((NCHUNK, CH), jnp.int32),
        pltpu.VMEM((NCHUNK, CH), jnp.int32),
        pltpu.VMEM((CH, H), jnp.float32),
        pltpu.VMEM_SHARED((N, H), jnp.float32),
        pltpu.SemaphoreType.DMA,
    ],
)


def _tc_in_body(x_ref, w_ref, b_ref, o_ref):
    o_ref[...] = jnp.tanh(
        jnp.dot(x_ref[...], w_ref[...], preferred_element_type=jnp.float32)
        + b_ref[...]
    )


def _tc_mid_body(a0_ref, a1_ref, d0_ref, d1_ref, w_ref, b_ref, o_ref):
    agg = a0_ref[0] + a1_ref[0]
    deg = jnp.maximum(d0_ref[0][:, :1] + d1_ref[0][:, :1], 1.0)
    t = jnp.dot(agg, w_ref[...], preferred_element_type=jnp.float32) / deg
    o_ref[...] = jnp.maximum(t + b_ref[...], 0.0)


def _tc_out_body(a0_ref, a1_ref, d0_ref, d1_ref, w_ref, b_ref, wo_ref, bo_ref,
                 o_ref, acc_ref):
    i = pl.program_id(0)

    @pl.when(i == 0)
    def _():
        acc_ref[...] = jnp.zeros_like(acc_ref)

    agg = a0_ref[0] + a1_ref[0]
    deg = jnp.maximum(d0_ref[0][:, :1] + d1_ref[0][:, :1], 1.0)
    t = jnp.dot(agg, w_ref[...], preferred_element_type=jnp.float32) / deg
    h = jnp.maximum(t + b_ref[...], 0.0)
    acc_ref[...] += jnp.sum(h, axis=0, keepdims=True)

    @pl.when(i == pl.num_programs(0) - 1)
    def _():
        y = jnp.sum(acc_ref[...] * wo_ref[...]) + bo_ref[0, 0]
        o_ref[...] = jnp.full((1, H), y, jnp.float32)


_tc_in = pl.pallas_call(
    _tc_in_body,
    grid=(GRID,),
    in_specs=[
        pl.BlockSpec((BLK, H), lambda i: (i, 0)),
        pl.BlockSpec((H, H), lambda i: (0, 0)),
        pl.BlockSpec((1, H), lambda i: (0, 0)),
    ],
    out_specs=pl.BlockSpec((BLK, H), lambda i: (i, 0)),
    out_shape=jax.ShapeDtypeStruct((N, H), jnp.float32),
)

_tc_mid = pl.pallas_call(
    _tc_mid_body,
    grid=(GRID,),
    in_specs=[
        pl.BlockSpec((1, BLK, H), lambda i: (0, i, 0)),
        pl.BlockSpec((1, BLK, H), lambda i: (1, i, 0)),
        pl.BlockSpec((1, BLK, 16), lambda i: (0, i, 0)),
        pl.BlockSpec((1, BLK, 16), lambda i: (1, i, 0)),
        pl.BlockSpec((H, H), lambda i: (0, 0)),
        pl.BlockSpec((1, H), lambda i: (0, 0)),
    ],
    out_specs=pl.BlockSpec((BLK, H), lambda i: (i, 0)),
    out_shape=jax.ShapeDtypeStruct((N, H), jnp.float32),
)

_tc_out = pl.pallas_call(
    _tc_out_body,
    grid=(GRID,),
    in_specs=[
        pl.BlockSpec((1, BLK, H), lambda i: (0, i, 0)),
        pl.BlockSpec((1, BLK, H), lambda i: (1, i, 0)),
        pl.BlockSpec((1, BLK, 16), lambda i: (0, i, 0)),
        pl.BlockSpec((1, BLK, 16), lambda i: (1, i, 0)),
        pl.BlockSpec((H, H), lambda i: (0, 0)),
        pl.BlockSpec((1, H), lambda i: (0, 0)),
        pl.BlockSpec((1, H), lambda i: (0, 0)),
        pl.BlockSpec((1, H), lambda i: (0, 0)),
    ],
    out_specs=pl.BlockSpec((1, H), lambda i: (0, 0)),
    out_shape=jax.ShapeDtypeStruct((1, H), jnp.float32),
    scratch_shapes=[pltpu.VMEM((1, H), jnp.float32)],
)


def kernel(x, edge_index, W_in, b_in, W1, b1, W2, b2, W_out, b_out):
    src = edge_index[0].astype(jnp.int32).reshape(NW, NCHUNK, CH)
    dst = edge_index[1].astype(jnp.int32).reshape(NW, NCHUNK, CH)
    zeros = jnp.zeros((N, H), jnp.float32)
    zdeg = jnp.zeros((N, 16), jnp.float32)
    ones = jnp.ones((CH, 16), jnp.float32)

    h0 = _tc_in(x, W_in, b_in.reshape(1, H))
    agg1, deg = _sc_agg_deg(h0, src, dst, zeros, zdeg, ones)
    h1 = _tc_mid(agg1, agg1, deg, deg, W1, b1.reshape(1, H))
    agg2 = _sc_agg(h1, src, dst, zeros)
    y = _tc_out(agg2, agg2, deg, deg, W2, b2.reshape(1, H),
                W_out.reshape(1, H),
                jnp.broadcast_to(b_out.reshape(1, 1), (1, H)))
    return y[0, 0]


# speedup vs baseline: 4.8313x; 4.8313x over previous
import jax
import jax.numpy as jnp
from jax import lax
from jax.experimental import pallas as pl
from jax.experimental.pallas import tpu as pltpu
from jax.experimental.pallas import tpu_sc as plsc

N = 10000
E = 320000
H = 128
NC = 2
NS = 16
NW = NC * NS
CH = 80
NCHUNK = E // (NW * CH)
STRIPE = 624
TAIL0 = NS * STRIPE
TAILN = N - TAIL0
BLK = 1000
GRID = N // BLK

_MESH = plsc.VectorSubcoreMesh(
    core_axis_name="c", subcore_axis_name="s", num_cores=NC, num_subcores=NS
)


def _sc_agg_body(h_hbm, src_hbm, dst_hbm, zeros_hbm,
                 out_agg,
                 src_v, dst_c, rows_v, acc, sem):
    c = lax.axis_index("c")
    s = lax.axis_index("s")
    w = c * NS + s
    r0 = s * STRIPE
    pltpu.sync_copy(zeros_hbm.at[pl.ds(r0, STRIPE), :], acc.at[pl.ds(r0, STRIPE), :])

    @pl.when(s == NS - 1)
    def _():
        pltpu.sync_copy(zeros_hbm.at[pl.ds(TAIL0, TAILN), :],
                        acc.at[pl.ds(TAIL0, TAILN), :])

    pltpu.sync_copy(src_hbm.at[w], src_v)
    plsc.subcore_barrier()

    def step(j, carry):
        pltpu.sync_copy(dst_hbm.at[w * NCHUNK + j], dst_c)
        pltpu.async_copy(h_hbm.at[src_v.at[j]], rows_v, sem).wait()
        pltpu.sync_copy(rows_v, acc.at[dst_c.at[0]], add=True)
        return carry

    lax.fori_loop(0, NCHUNK, step, 0)
    plsc.subcore_barrier()
    pltpu.sync_copy(acc.at[pl.ds(r0, STRIPE), :], out_agg.at[c, pl.ds(r0, STRIPE), :])

    @pl.when(s == NS - 1)
    def _():
        pltpu.sync_copy(acc.at[pl.ds(TAIL0, TAILN), :],
                        out_agg.at[c, pl.ds(TAIL0, TAILN), :])


def _sc_deg_body(dst_hbm, zeros_hbm, ones_hbm,
                 out_deg,
                 dst_c, ones_v, accd):
    c = lax.axis_index("c")
    s = lax.axis_index("s")
    w = c * NS + s
    r0 = s * STRIPE
    pltpu.sync_copy(zeros_hbm.at[pl.ds(r0, STRIPE), :], accd.at[pl.ds(r0, STRIPE), :])

    @pl.when(s == NS - 1)
    def _():
        pltpu.sync_copy(zeros_hbm.at[pl.ds(TAIL0, TAILN), :],
                        accd.at[pl.ds(TAIL0, TAILN), :])

    pltpu.sync_copy(ones_hbm, ones_v)
    plsc.subcore_barrier()

    def step(j, carry):
        pltpu.sync_copy(dst_hbm.at[w * NCHUNK + j], dst_c)
        pltpu.sync_copy(ones_v, accd.at[dst_c.at[0]], add=True)
        return carry

    lax.fori_loop(0, NCHUNK, step, 0)
    plsc.subcore_barrier()
    pltpu.sync_copy(accd.at[pl.ds(r0, STRIPE), :], out_deg.at[c, pl.ds(r0, STRIPE), :])

    @pl.when(s == NS - 1)
    def _():
        pltpu.sync_copy(accd.at[pl.ds(TAIL0, TAILN), :],
                        out_deg.at[c, pl.ds(TAIL0, TAILN), :])


_sc_agg = pl.kernel(
    _sc_agg_body,
    out_type=jax.ShapeDtypeStruct((NC, N, H), jnp.float32),
    mesh=_MESH,
    scratch_types=[
        pltpu.VMEM((NCHUNK, CH), jnp.int32),
        pltpu.VMEM((1, CH), jnp.int32),
        pltpu.VMEM((CH, H), jnp.float32),
        pltpu.VMEM_SHARED((N, H), jnp.float32),
        pltpu.SemaphoreType.DMA,
    ],
)

_sc_deg = pl.kernel(
    _sc_deg_body,
    out_type=jax.ShapeDtypeStruct((NC, N, H), jnp.float32),
    mesh=_MESH,
    scratch_types=[
        pltpu.VMEM((1, CH), jnp.int32),
        pltpu.VMEM((CH, H), jnp.float32),
        pltpu.VMEM_SHARED((N, H), jnp.float32),
    ],
)


def _tc_in_body(x_ref, w_ref, b_ref, o_ref):
    o_ref[...] = jnp.tanh(
        jnp.dot(x_ref[...], w_ref[...], preferred_element_type=jnp.float32)
        + b_ref[...]
    )


def _tc_mid_body(a0_ref, a1_ref, d0_ref, d1_ref, w_ref, b_ref, o_ref):
    agg = a0_ref[0] + a1_ref[0]
    deg = jnp.maximum(d0_ref[0][:, :1] + d1_ref[0][:, :1], 1.0)
    t = jnp.dot(agg, w_ref[...], preferred_element_type=jnp.float32) / deg
    o_ref[...] = jnp.maximum(t + b_ref[...], 0.0)


def _tc_out_body(a0_ref, a1_ref, d0_ref, d1_ref, w_ref, b_ref, wo_ref, bo_ref,
                 o_ref, acc_ref):
    i = pl.program_id(0)

    @pl.when(i == 0)
    def _():
        acc_ref[...] = jnp.zeros_like(acc_ref)

    agg = a0_ref[0] + a1_ref[0]
    deg = jnp.maximum(d0_ref[0][:, :1] + d1_ref[0][:, :1], 1.0)
    t = jnp.dot(agg, w_ref[...], preferred_element_type=jnp.float32) / deg
    h = jnp.maximum(t + b_ref[...], 0.0)
    acc_ref[...] += jnp.sum(h, axis=0, keepdims=True)

    @pl.when(i == pl.num_programs(0) - 1)
    def _():
        y = jnp.sum(acc_ref[...] * wo_ref[...]) + bo_ref[0, 0]
        o_ref[...] = jnp.full((1, H), y, jnp.float32)


_tc_in = pl.pallas_call(
    _tc_in_body,
    grid=(GRID,),
    in_specs=[
        pl.BlockSpec((BLK, H), lambda i: (i, 0)),
        pl.BlockSpec((H, H), lambda i: (0, 0)),
        pl.BlockSpec((1, H), lambda i: (0, 0)),
    ],
    out_specs=pl.BlockSpec((BLK, H), lambda i: (i, 0)),
    out_shape=jax.ShapeDtypeStruct((N, H), jnp.float32),
)

_tc_mid = pl.pallas_call(
    _tc_mid_body,
    grid=(GRID,),
    in_specs=[
        pl.BlockSpec((1, BLK, H), lambda i: (0, i, 0)),
        pl.BlockSpec((1, BLK, H), lambda i: (1, i, 0)),
        pl.BlockSpec((1, BLK, H), lambda i: (0, i, 0)),
        pl.BlockSpec((1, BLK, H), lambda i: (1, i, 0)),
        pl.BlockSpec((H, H), lambda i: (0, 0)),
        pl.BlockSpec((1, H), lambda i: (0, 0)),
    ],
    out_specs=pl.BlockSpec((BLK, H), lambda i: (i, 0)),
    out_shape=jax.ShapeDtypeStruct((N, H), jnp.float32),
)

_tc_out = pl.pallas_call(
    _tc_out_body,
    grid=(GRID,),
    in_specs=[
        pl.BlockSpec((1, BLK, H), lambda i: (0, i, 0)),
        pl.BlockSpec((1, BLK, H), lambda i: (1, i, 0)),
        pl.BlockSpec((1, BLK, H), lambda i: (0, i, 0)),
        pl.BlockSpec((1, BLK, H), lambda i: (1, i, 0)),
        pl.BlockSpec((H, H), lambda i: (0, 0)),
        pl.BlockSpec((1, H), lambda i: (0, 0)),
        pl.BlockSpec((1, H), lambda i: (0, 0)),
        pl.BlockSpec((1, H), lambda i: (0, 0)),
    ],
    out_specs=pl.BlockSpec((1, H), lambda i: (0, 0)),
    out_shape=jax.ShapeDtypeStruct((1, H), jnp.float32),
    scratch_shapes=[pltpu.VMEM((1, H), jnp.float32)],
)


def kernel(x, edge_index, W_in, b_in, W1, b1, W2, b2, W_out, b_out):
    src = edge_index[0].astype(jnp.int32).reshape(NW, NCHUNK, CH)
    dst = edge_index[1].astype(jnp.int32).reshape(NW * NCHUNK, 1, CH)
    zeros = jnp.zeros((N, H), jnp.float32)
    ones = jnp.ones((CH, H), jnp.float32)

    deg = _sc_deg(dst, zeros, ones)
    h0 = _tc_in(x, W_in, b_in.reshape(1, H))
    agg1 = _sc_agg(h0, src, dst, zeros)
    h1 = _tc_mid(agg1, agg1, deg, deg, W1, b1.reshape(1, H))
    agg2 = _sc_agg(h1, src, dst, zeros)
    y = _tc_out(agg2, agg2, deg, deg, W2, b2.reshape(1, H),
                W_out.reshape(1, H),
                jnp.broadcast_to(b_out.reshape(1, 1), (1, H)))
    return y[0, 0]


# speedup vs baseline: 7.1433x; 1.4786x over previous
import jax
import jax.numpy as jnp
from jax import lax
from jax.experimental import pallas as pl
from jax.experimental.pallas import tpu as pltpu
from jax.experimental.pallas import tpu_sc as plsc

N = 10000
E = 320000
H = 128
NC = 2
NS = 16
NW = NC * NS
CH = 40
NCHUNK = E // (NW * CH)
UN = 5
GB = 25
NG = NCHUNK // GB
NI = GB // UN
STRIPE = 624
TAIL0 = NS * STRIPE
TAILN = N - TAIL0
BLK = 1000
GRID = N // BLK

_MESH = plsc.VectorSubcoreMesh(
    core_axis_name="c", subcore_axis_name="s", num_cores=NC, num_subcores=NS
)


def _sc_agg_body(h_hbm, src_hbm, dst_hbm, zeros_hbm,
                 out_agg,
                 src_c, dst_c, rows_v, acc, sem, ssem):
    c = lax.axis_index("c")
    s = lax.axis_index("s")
    w = c * NS + s
    r0 = s * STRIPE
    pltpu.sync_copy(zeros_hbm.at[pl.ds(r0, STRIPE), :], acc.at[pl.ds(r0, STRIPE), :])

    @pl.when(s == NS - 1)
    def _():
        pltpu.sync_copy(zeros_hbm.at[pl.ds(TAIL0, TAILN), :],
                        acc.at[pl.ds(TAIL0, TAILN), :])

    plsc.subcore_barrier()

    def group(g, carry):
        base = w * NCHUNK + g * GB
        pltpu.sync_copy(src_hbm.at[pl.ds(base, GB)], src_c)
        pltpu.sync_copy(dst_hbm.at[pl.ds(base, GB)], dst_c)

        def step(k, carry2):
            j0 = k * UN
            gathers = [
                pltpu.async_copy(h_hbm.at[src_c.at[j0 + b, 0]], rows_v.at[b], sem)
                for b in range(UN)
            ]
            for gg in gathers:
                gg.wait()
            scatters = [
                pltpu.async_copy(rows_v.at[b], acc.at[dst_c.at[j0 + b, 0]],
                                 ssem, add=True)
                for b in range(UN)
            ]
            for sc in scatters:
                sc.wait()
            return carry2

        lax.fori_loop(0, NI, step, 0)
        return carry

    lax.fori_loop(0, NG, group, 0)
    plsc.subcore_barrier()
    pltpu.sync_copy(acc.at[pl.ds(r0, STRIPE), :], out_agg.at[c, pl.ds(r0, STRIPE), :])

    @pl.when(s == NS - 1)
    def _():
        pltpu.sync_copy(acc.at[pl.ds(TAIL0, TAILN), :],
                        out_agg.at[c, pl.ds(TAIL0, TAILN), :])


def _sc_deg_body(dst_hbm, zeros_hbm, ones_hbm,
                 out_deg,
                 dst_c, ones_v, accd, ssem):
    c = lax.axis_index("c")
    s = lax.axis_index("s")
    w = c * NS + s
    r0 = s * STRIPE
    pltpu.sync_copy(zeros_hbm.at[pl.ds(r0, STRIPE), :], accd.at[pl.ds(r0, STRIPE), :])

    @pl.when(s == NS - 1)
    def _():
        pltpu.sync_copy(zeros_hbm.at[pl.ds(TAIL0, TAILN), :],
                        accd.at[pl.ds(TAIL0, TAILN), :])

    pltpu.sync_copy(ones_hbm, ones_v)
    plsc.subcore_barrier()

    def group(g, carry):
        base = w * NCHUNK + g * GB
        pltpu.sync_copy(dst_hbm.at[pl.ds(base, GB)], dst_c)

        def step(k, carry2):
            j0 = k * UN
            scatters = [
                pltpu.async_copy(ones_v, accd.at[dst_c.at[j0 + b, 0]],
                                 ssem, add=True)
                for b in range(UN)
            ]
            for sc in scatters:
                sc.wait()
            return carry2

        lax.fori_loop(0, NI, step, 0)
        return carry

    lax.fori_loop(0, NG, group, 0)
    plsc.subcore_barrier()
    pltpu.sync_copy(accd.at[pl.ds(r0, STRIPE), :], out_deg.at[c, pl.ds(r0, STRIPE), :])

    @pl.when(s == NS - 1)
    def _():
        pltpu.sync_copy(accd.at[pl.ds(TAIL0, TAILN), :],
                        out_deg.at[c, pl.ds(TAIL0, TAILN), :])


_sc_agg = pl.kernel(
    _sc_agg_body,
    out_type=jax.ShapeDtypeStruct((NC, N, H), jnp.float32),
    mesh=_MESH,
    scratch_types=[
        pltpu.VMEM((GB, 1, CH), jnp.int32),
        pltpu.VMEM((GB, 1, CH), jnp.int32),
        pltpu.VMEM((UN, CH, H), jnp.float32),
        pltpu.VMEM_SHARED((N, H), jnp.float32),
        pltpu.SemaphoreType.DMA,
        pltpu.SemaphoreType.DMA,
    ],
)

_sc_deg = pl.kernel(
    _sc_deg_body,
    out_type=jax.ShapeDtypeStruct((NC, N, H), jnp.float32),
    mesh=_MESH,
    scratch_types=[
        pltpu.VMEM((GB, 1, CH), jnp.int32),
        pltpu.VMEM((CH, H), jnp.float32),
        pltpu.VMEM_SHARED((N, H), jnp.float32),
        pltpu.SemaphoreType.DMA,
    ],
)


def _tc_in_body(x_ref, w_ref, b_ref, o_ref):
    o_ref[...] = jnp.tanh(
        jnp.dot(x_ref[...], w_ref[...], preferred_element_type=jnp.float32)
        + b_ref[...]
    )


def _tc_mid_body(a0_ref, a1_ref, d0_ref, d1_ref, w_ref, b_ref, o_ref):
    agg = a0_ref[0] + a1_ref[0]
    deg = jnp.maximum(d0_ref[0][:, :1] + d1_ref[0][:, :1], 1.0)
    t = jnp.dot(agg, w_ref[...], preferred_element_type=jnp.float32) / deg
    o_ref[...] = jnp.maximum(t + b_ref[...], 0.0)


def _tc_out_body(a0_ref, a1_ref, d0_ref, d1_ref, w_ref, b_ref, wo_ref, bo_ref,
                 o_ref, acc_ref):
    i = pl.program_id(0)

    @pl.when(i == 0)
    def _():
        acc_ref[...] = jnp.zeros_like(acc_ref)

    agg = a0_ref[0] + a1_ref[0]
    deg = jnp.maximum(d0_ref[0][:, :1] + d1_ref[0][:, :1], 1.0)
    t = jnp.dot(agg, w_ref[...], preferred_element_type=jnp.float32) / deg
    h = jnp.maximum(t + b_ref[...], 0.0)
    acc_ref[...] += jnp.sum(h, axis=0, keepdims=True)

    @pl.when(i == pl.num_programs(0) - 1)
    def _():
        y = jnp.sum(acc_ref[...] * wo_ref[...]) + bo_ref[0, 0]
        o_ref[...] = jnp.full((1, H), y, jnp.float32)


_tc_in = pl.pallas_call(
    _tc_in_body,
    grid=(GRID,),
    in_specs=[
        pl.BlockSpec((BLK, H), lambda i: (i, 0)),
        pl.BlockSpec((H, H), lambda i: (0, 0)),
        pl.BlockSpec((1, H), lambda i: (0, 0)),
    ],
    out_specs=pl.BlockSpec((BLK, H), lambda i: (i, 0)),
    out_shape=jax.ShapeDtypeStruct((N, H), jnp.float32),
)

_tc_mid = pl.pallas_call(
    _tc_mid_body,
    grid=(GRID,),
    in_specs=[
        pl.BlockSpec((1, BLK, H), lambda i: (0, i, 0)),
        pl.BlockSpec((1, BLK, H), lambda i: (1, i, 0)),
        pl.BlockSpec((1, BLK, H), lambda i: (0, i, 0)),
        pl.BlockSpec((1, BLK, H), lambda i: (1, i, 0)),
        pl.BlockSpec((H, H), lambda i: (0, 0)),
        pl.BlockSpec((1, H), lambda i: (0, 0)),
    ],
    out_specs=pl.BlockSpec((BLK, H), lambda i: (i, 0)),
    out_shape=jax.ShapeDtypeStruct((N, H), jnp.float32),
)

_tc_out = pl.pallas_call(
    _tc_out_body,
    grid=(GRID,),
    in_specs=[
        pl.BlockSpec((1, BLK, H), lambda i: (0, i, 0)),
        pl.BlockSpec((1, BLK, H), lambda i: (1, i, 0)),
        pl.BlockSpec((1, BLK, H), lambda i: (0, i, 0)),
        pl.BlockSpec((1, BLK, H), lambda i: (1, i, 0)),
        pl.BlockSpec((H, H), lambda i: (0, 0)),
        pl.BlockSpec((1, H), lambda i: (0, 0)),
        pl.BlockSpec((1, H), lambda i: (0, 0)),
        pl.BlockSpec((1, H), lambda i: (0, 0)),
    ],
    out_specs=pl.BlockSpec((1, H), lambda i: (0, 0)),
    out_shape=jax.ShapeDtypeStruct((1, H), jnp.float32),
    scratch_shapes=[pltpu.VMEM((1, H), jnp.float32)],
)


def kernel(x, edge_index, W_in, b_in, W1, b1, W2, b2, W_out, b_out):
    src = edge_index[0].astype(jnp.int32).reshape(NW * NCHUNK, 1, CH)
    dst = edge_index[1].astype(jnp.int32).reshape(NW * NCHUNK, 1, CH)
    zeros = jnp.zeros((N, H), jnp.float32)
    ones = jnp.ones((CH, H), jnp.float32)

    deg = _sc_deg(dst, zeros, ones)
    h0 = _tc_in(x, W_in, b_in.reshape(1, H))
    agg1 = _sc_agg(h0, src, dst, zeros)
    h1 = _tc_mid(agg1, agg1, deg, deg, W1, b1.reshape(1, H))
    agg2 = _sc_agg(h1, src, dst, zeros)
    y = _tc_out(agg2, agg2, deg, deg, W2, b2.reshape(1, H),
                W_out.reshape(1, H),
                jnp.broadcast_to(b_out.reshape(1, 1), (1, H)))
    return y[0, 0]


# speedup vs baseline: 8.6505x; 1.2110x over previous
import jax
import jax.numpy as jnp
from jax import lax
from jax.experimental import pallas as pl
from jax.experimental.pallas import tpu as pltpu
from jax.experimental.pallas import tpu_sc as plsc

N = 10000
E = 320000
H = 128
NC = 2
NS = 16
NW = NC * NS
CH = 80
NCHUNK = E // (NW * CH)
GB = 25
NG = NCHUNK // GB
GPAIR = (GB - 1) // 2
DCH = 40
DNCHUNK = E // (NW * DCH)
DUN = 5
DGB = 25
DNG = DNCHUNK // DGB
DNI = DGB // DUN
STRIPE = 624
TAIL0 = NS * STRIPE
TAILN = N - TAIL0
BLK = 1000
GRID = N // BLK

_MESH = plsc.VectorSubcoreMesh(
    core_axis_name="c", subcore_axis_name="s", num_cores=NC, num_subcores=NS
)


def _sc_agg_body(h_hbm, src_hbm, dst_hbm, zeros_hbm,
                 out_agg,
                 src_st, dst_st, rows_a, rows_b, acc, ga, gb, sa, sb):
    c = lax.axis_index("c")
    s = lax.axis_index("s")
    w = c * NS + s
    r0 = s * STRIPE
    pltpu.sync_copy(zeros_hbm.at[pl.ds(r0, STRIPE), :], acc.at[pl.ds(r0, STRIPE), :])

    @pl.when(s == NS - 1)
    def _():
        pltpu.sync_copy(zeros_hbm.at[pl.ds(TAIL0, TAILN), :],
                        acc.at[pl.ds(TAIL0, TAILN), :])

    plsc.subcore_barrier()

    def group(g, carry):
        base = w * NCHUNK + g * GB
        pltpu.sync_copy(src_hbm.at[pl.ds(base, GB)], src_st)
        pltpu.sync_copy(dst_hbm.at[pl.ds(base, GB)], dst_st)
        pltpu.async_copy(h_hbm.at[src_st.at[0, 0]], rows_a, ga)

        def pair(i, carry2):
            t0 = 2 * i
            pltpu.async_copy(h_hbm.at[src_st.at[t0 + 1, 0]], rows_b, gb)
            pltpu.make_async_copy(h_hbm.at[src_st.at[t0, 0]], rows_a, ga).wait()
            pltpu.async_copy(rows_a, acc.at[dst_st.at[t0, 0]], sa, add=True).wait()
            pltpu.async_copy(h_hbm.at[src_st.at[t0 + 2, 0]], rows_a, ga)
            pltpu.make_async_copy(h_hbm.at[src_st.at[t0 + 1, 0]], rows_b, gb).wait()
            pltpu.async_copy(rows_b, acc.at[dst_st.at[t0 + 1, 0]], sb, add=True).wait()
            return carry2

        lax.fori_loop(0, GPAIR, pair, 0)
        pltpu.make_async_copy(h_hbm.at[src_st.at[GB - 1, 0]], rows_a, ga).wait()
        pltpu.sync_copy(rows_a, acc.at[dst_st.at[GB - 1, 0]], add=True)
        return carry

    lax.fori_loop(0, NG, group, 0)
    plsc.subcore_barrier()
    pltpu.sync_copy(acc.at[pl.ds(r0, STRIPE), :], out_agg.at[c, pl.ds(r0, STRIPE), :])

    @pl.when(s == NS - 1)
    def _():
        pltpu.sync_copy(acc.at[pl.ds(TAIL0, TAILN), :],
                        out_agg.at[c, pl.ds(TAIL0, TAILN), :])


def _sc_deg_body(dst_hbm, zeros_hbm, ones_hbm,
                 out_deg,
                 dst_c, ones_v, accd, ssem):
    c = lax.axis_index("c")
    s = lax.axis_index("s")
    w = c * NS + s
    r0 = s * STRIPE
    pltpu.sync_copy(zeros_hbm.at[pl.ds(r0, STRIPE), :], accd.at[pl.ds(r0, STRIPE), :])

    @pl.when(s == NS - 1)
    def _():
        pltpu.sync_copy(zeros_hbm.at[pl.ds(TAIL0, TAILN), :],
                        accd.at[pl.ds(TAIL0, TAILN), :])

    pltpu.sync_copy(ones_hbm, ones_v)
    plsc.subcore_barrier()

    def group(g, carry):
        base = w * DNCHUNK + g * DGB
        pltpu.sync_copy(dst_hbm.at[pl.ds(base, DGB)], dst_c)

        def step(k, carry2):
            j0 = k * DUN
            scatters = [
                pltpu.async_copy(ones_v, accd.at[dst_c.at[j0 + b, 0]],
                                 ssem, add=True)
                for b in range(DUN)
            ]
            for sc in scatters:
                sc.wait()
            return carry2

        lax.fori_loop(0, DNI, step, 0)
        return carry

    lax.fori_loop(0, DNG, group, 0)
    plsc.subcore_barrier()
    pltpu.sync_copy(accd.at[pl.ds(r0, STRIPE), :], out_deg.at[c, pl.ds(r0, STRIPE), :])

    @pl.when(s == NS - 1)
    def _():
        pltpu.sync_copy(accd.at[pl.ds(TAIL0, TAILN), :],
                        out_deg.at[c, pl.ds(TAIL0, TAILN), :])


_sc_agg = pl.kernel(
    _sc_agg_body,
    out_type=jax.ShapeDtypeStruct((NC, N, H), jnp.float32),
    mesh=_MESH,
    scratch_types=[
        pltpu.VMEM((GB, 1, CH), jnp.int32),
        pltpu.VMEM((GB, 1, CH), jnp.int32),
        pltpu.VMEM((CH, H), jnp.float32),
        pltpu.VMEM((CH, H), jnp.float32),
        pltpu.VMEM_SHARED((N, H), jnp.float32),
        pltpu.SemaphoreType.DMA,
        pltpu.SemaphoreType.DMA,
        pltpu.SemaphoreType.DMA,
        pltpu.SemaphoreType.DMA,
    ],
)

_sc_deg = pl.kernel(
    _sc_deg_body,
    out_type=jax.ShapeDtypeStruct((NC, N, H), jnp.float32),
    mesh=_MESH,
    scratch_types=[
        pltpu.VMEM((DGB, 1, DCH), jnp.int32),
        pltpu.VMEM((DCH, H), jnp.float32),
        pltpu.VMEM_SHARED((N, H), jnp.float32),
        pltpu.SemaphoreType.DMA,
    ],
)


def _tc_in_body(x_ref, w_ref, b_ref, o_ref):
    o_ref[...] = jnp.tanh(
        jnp.dot(x_ref[...], w_ref[...], preferred_element_type=jnp.float32)
        + b_ref[...]
    )


def _tc_mid_body(a0_ref, a1_ref, d0_ref, d1_ref, w_ref, b_ref, o_ref):
    agg = a0_ref[0] + a1_ref[0]
    deg = jnp.maximum(d0_ref[0][:, :1] + d1_ref[0][:, :1], 1.0)
    t = jnp.dot(agg, w_ref[...], preferred_element_type=jnp.float32) / deg
    o_ref[...] = jnp.maximum(t + b_ref[...], 0.0)


def _tc_out_body(a0_ref, a1_ref, d0_ref, d1_ref, w_ref, b_ref, wo_ref, bo_ref,
                 o_ref, acc_ref):
    i = pl.program_id(0)

    @pl.when(i == 0)
    def _():
        acc_ref[...] = jnp.zeros_like(acc_ref)

    agg = a0_ref[0] + a1_ref[0]
    deg = jnp.maximum(d0_ref[0][:, :1] + d1_ref[0][:, :1], 1.0)
    t = jnp.dot(agg, w_ref[...], preferred_element_type=jnp.float32) / deg
    h = jnp.maximum(t + b_ref[...], 0.0)
    acc_ref[...] += jnp.sum(h, axis=0, keepdims=True)

    @pl.when(i == pl.num_programs(0) - 1)
    def _():
        y = jnp.sum(acc_ref[...] * wo_ref[...]) + bo_ref[0, 0]
        o_ref[...] = jnp.full((1, H), y, jnp.float32)


_tc_in = pl.pallas_call(
    _tc_in_body,
    grid=(GRID,),
    in_specs=[
        pl.BlockSpec((BLK, H), lambda i: (i, 0)),
        pl.BlockSpec((H, H), lambda i: (0, 0)),
        pl.BlockSpec((1, H), lambda i: (0, 0)),
    ],
    out_specs=pl.BlockSpec((BLK, H), lambda i: (i, 0)),
    out_shape=jax.ShapeDtypeStruct((N, H), jnp.float32),
)

_tc_mid = pl.pallas_call(
    _tc_mid_body,
    grid=(GRID,),
    in_specs=[
        pl.BlockSpec((1, BLK, H), lambda i: (0, i, 0)),
        pl.BlockSpec((1, BLK, H), lambda i: (1, i, 0)),
        pl.BlockSpec((1, BLK, H), lambda i: (0, i, 0)),
        pl.BlockSpec((1, BLK, H), lambda i: (1, i, 0)),
        pl.BlockSpec((H, H), lambda i: (0, 0)),
        pl.BlockSpec((1, H), lambda i: (0, 0)),
    ],
    out_specs=pl.BlockSpec((BLK, H), lambda i: (i, 0)),
    out_shape=jax.ShapeDtypeStruct((N, H), jnp.float32),
)

_tc_out = pl.pallas_call(
    _tc_out_body,
    grid=(GRID,),
    in_specs=[
        pl.BlockSpec((1, BLK, H), lambda i: (0, i, 0)),
        pl.BlockSpec((1, BLK, H), lambda i: (1, i, 0)),
        pl.BlockSpec((1, BLK, H), lambda i: (0, i, 0)),
        pl.BlockSpec((1, BLK, H), lambda i: (1, i, 0)),
        pl.BlockSpec((H, H), lambda i: (0, 0)),
        pl.BlockSpec((1, H), lambda i: (0, 0)),
        pl.BlockSpec((1, H), lambda i: (0, 0)),
        pl.BlockSpec((1, H), lambda i: (0, 0)),
    ],
    out_specs=pl.BlockSpec((1, H), lambda i: (0, 0)),
    out_shape=jax.ShapeDtypeStruct((1, H), jnp.float32),
    scratch_shapes=[pltpu.VMEM((1, H), jnp.float32)],
)


def kernel(x, edge_index, W_in, b_in, W1, b1, W2, b2, W_out, b_out):
    src = edge_index[0].astype(jnp.int32).reshape(NW * NCHUNK, 1, CH)
    dst = edge_index[1].astype(jnp.int32).reshape(NW * NCHUNK, 1, CH)
    dst_d = edge_index[1].astype(jnp.int32).reshape(NW * DNCHUNK, 1, DCH)
    zeros = jnp.zeros((N, H), jnp.float32)
    ones = jnp.ones((DCH, H), jnp.float32)

    deg = _sc_deg(dst_d, zeros, ones)
    h0 = _tc_in(x, W_in, b_in.reshape(1, H))
    agg1 = _sc_agg(h0, src, dst, zeros)
    h1 = _tc_mid(agg1, agg1, deg, deg, W1, b1.reshape(1, H))
    agg2 = _sc_agg(h1, src, dst, zeros)
    y = _tc_out(agg2, agg2, deg, deg, W2, b2.reshape(1, H),
                W_out.reshape(1, H),
                jnp.broadcast_to(b_out.reshape(1, 1), (1, H)))
    return y[0, 0]


# speedup vs baseline: 8.9142x; 1.0305x over previous
import jax
import jax.numpy as jnp
from jax import lax
from jax.experimental import pallas as pl
from jax.experimental.pallas import tpu as pltpu
from jax.experimental.pallas import tpu_sc as plsc

N = 10000
E = 320000
H = 128
NC = 2
NS = 16
NW = NC * NS
CH = 40
NCHUNK = E // (NW * CH)
NB = 5
GB = 25
NG = NCHUNK // GB
NBODY = GB // NB - 1
DCH = 40
DNCHUNK = E // (NW * DCH)
DUN = 5
DGB = 25
DNG = DNCHUNK // DGB
DNI = DGB // DUN
STRIPE = 624
TAIL0 = NS * STRIPE
TAILN = N - TAIL0
BLK = 1000
GRID = N // BLK

_MESH = plsc.VectorSubcoreMesh(
    core_axis_name="c", subcore_axis_name="s", num_cores=NC, num_subcores=NS
)


def _sc_agg_body(h_hbm, src_hbm, dst_hbm, zeros_hbm,
                 out_agg,
                 idx_st, rows_v, acc,
                 g0, g1, g2, g3, g4, ssem):
    c = lax.axis_index("c")
    s = lax.axis_index("s")
    w = c * NS + s
    r0 = s * STRIPE
    pltpu.sync_copy(zeros_hbm.at[pl.ds(r0, STRIPE), :], acc.at[pl.ds(r0, STRIPE), :])

    @pl.when(s == NS - 1)
    def _():
        pltpu.sync_copy(zeros_hbm.at[pl.ds(TAIL0, TAILN), :],
                        acc.at[pl.ds(TAIL0, TAILN), :])

    plsc.subcore_barrier()

    rows = [rows_v.at[k] for k in range(NB)]
    gsem = [g0, g1, g2, g3, g4]

    def group(g, carry):
        base = w * NCHUNK + g * GB
        pltpu.sync_copy(src_hbm.at[pl.ds(base, GB)], idx_st.at[0])
        pltpu.sync_copy(dst_hbm.at[pl.ds(base, GB)], idx_st.at[1])
        for k in range(NB):
            pltpu.async_copy(h_hbm.at[idx_st.at[0, k, 0]], rows[k], gsem[k])

        def body(i, carry2):
            t0 = NB * i
            for k in range(NB):
                lc = t0 + k
                pltpu.make_async_copy(
                    h_hbm.at[idx_st.at[0, lc, 0]], rows[k], gsem[k]).wait()
                pltpu.async_copy(
                    rows[k], acc.at[idx_st.at[1, lc, 0]], ssem, add=True).wait()
                pltpu.async_copy(h_hbm.at[idx_st.at[0, lc + NB, 0]], rows[k],
                                 gsem[k])
            return carry2

        lax.fori_loop(0, NBODY, body, 0)
        for k in range(NB):
            lc = NB * NBODY + k
            pltpu.make_async_copy(
                h_hbm.at[idx_st.at[0, lc, 0]], rows[k], gsem[k]).wait()
            pltpu.async_copy(
                rows[k], acc.at[idx_st.at[1, lc, 0]], ssem, add=True).wait()
        return carry

    lax.fori_loop(0, NG, group, 0)
    plsc.subcore_barrier()
    pltpu.sync_copy(acc.at[pl.ds(r0, STRIPE), :], out_agg.at[c, pl.ds(r0, STRIPE), :])

    @pl.when(s == NS - 1)
    def _():
        pltpu.sync_copy(acc.at[pl.ds(TAIL0, TAILN), :],
                        out_agg.at[c, pl.ds(TAIL0, TAILN), :])


def _sc_deg_body(dst_hbm, zeros_hbm, ones_hbm,
                 out_deg,
                 dst_c, ones_v, accd, ssem):
    c = lax.axis_index("c")
    s = lax.axis_index("s")
    w = c * NS + s
    r0 = s * STRIPE
    pltpu.sync_copy(zeros_hbm.at[pl.ds(r0, STRIPE), :], accd.at[pl.ds(r0, STRIPE), :])

    @pl.when(s == NS - 1)
    def _():
        pltpu.sync_copy(zeros_hbm.at[pl.ds(TAIL0, TAILN), :],
                        accd.at[pl.ds(TAIL0, TAILN), :])

    pltpu.sync_copy(ones_hbm, ones_v)
    plsc.subcore_barrier()

    def group(g, carry):
        base = w * DNCHUNK + g * DGB
        pltpu.sync_copy(dst_hbm.at[pl.ds(base, DGB)], dst_c)

        def step(k, carry2):
            j0 = k * DUN
            scatters = [
                pltpu.async_copy(ones_v, accd.at[dst_c.at[j0 + b, 0]],
                                 ssem, add=True)
                for b in range(DUN)
            ]
            for sc in scatters:
                sc.wait()
            return carry2

        lax.fori_loop(0, DNI, step, 0)
        return carry

    lax.fori_loop(0, DNG, group, 0)
    plsc.subcore_barrier()
    pltpu.sync_copy(accd.at[pl.ds(r0, STRIPE), :], out_deg.at[c, pl.ds(r0, STRIPE), :])

    @pl.when(s == NS - 1)
    def _():
        pltpu.sync_copy(accd.at[pl.ds(TAIL0, TAILN), :],
                        out_deg.at[c, pl.ds(TAIL0, TAILN), :])


_sc_agg = pl.kernel(
    _sc_agg_body,
    out_type=jax.ShapeDtypeStruct((NC, N, H), jnp.float32),
    mesh=_MESH,
    scratch_types=(
        [pltpu.VMEM((2, GB, 1, CH), jnp.int32)]
        + [pltpu.VMEM((NB, CH, H), jnp.float32)]
        + [pltpu.VMEM_SHARED((N, H), jnp.float32)]
        + [pltpu.SemaphoreType.DMA] * (NB + 1)
    ),
)

_sc_deg = pl.kernel(
    _sc_deg_body,
    out_type=jax.ShapeDtypeStruct((NC, N, H), jnp.float32),
    mesh=_MESH,
    scratch_types=[
        pltpu.VMEM((DGB, 1, DCH), jnp.int32),
        pltpu.VMEM((DCH, H), jnp.float32),
        pltpu.VMEM_SHARED((N, H), jnp.float32),
        pltpu.SemaphoreType.DMA,
    ],
)


def _tc_in_body(x_ref, w_ref, b_ref, o_ref):
    o_ref[...] = jnp.tanh(
        jnp.dot(x_ref[...], w_ref[...], preferred_element_type=jnp.float32)
        + b_ref[...]
    )


def _tc_mid_body(a0_ref, a1_ref, d0_ref, d1_ref, w_ref, b_ref, o_ref):
    agg = a0_ref[0] + a1_ref[0]
    deg = jnp.maximum(d0_ref[0][:, :1] + d1_ref[0][:, :1], 1.0)
    t = jnp.dot(agg, w_ref[...], preferred_element_type=jnp.float32) / deg
    o_ref[...] = jnp.maximum(t + b_ref[...], 0.0)


def _tc_out_body(a0_ref, a1_ref, d0_ref, d1_ref, w_ref, b_ref, wo_ref, bo_ref,
                 o_ref, acc_ref):
    i = pl.program_id(0)

    @pl.when(i == 0)
    def _():
        acc_ref[...] = jnp.zeros_like(acc_ref)

    agg = a0_ref[0] + a1_ref[0]
    deg = jnp.maximum(d0_ref[0][:, :1] + d1_ref[0][:, :1], 1.0)
    t = jnp.dot(agg, w_ref[...], preferred_element_type=jnp.float32) / deg
    h = jnp.maximum(t + b_ref[...], 0.0)
    acc_ref[...] += jnp.sum(h, axis=0, keepdims=True)

    @pl.when(i == pl.num_programs(0) - 1)
    def _():
        y = jnp.sum(acc_ref[...] * wo_ref[...]) + bo_ref[0, 0]
        o_ref[...] = jnp.full((1, H), y, jnp.float32)


_tc_in = pl.pallas_call(
    _tc_in_body,
    grid=(GRID,),
    in_specs=[
        pl.BlockSpec((BLK, H), lambda i: (i, 0)),
        pl.BlockSpec((H, H), lambda i: (0, 0)),
        pl.BlockSpec((1, H), lambda i: (0, 0)),
    ],
    out_specs=pl.BlockSpec((BLK, H), lambda i: (i, 0)),
    out_shape=jax.ShapeDtypeStruct((N, H), jnp.float32),
)

_tc_mid = pl.pallas_call(
    _tc_mid_body,
    grid=(GRID,),
    in_specs=[
        pl.BlockSpec((1, BLK, H), lambda i: (0, i, 0)),
        pl.BlockSpec((1, BLK, H), lambda i: (1, i, 0)),
        pl.BlockSpec((1, BLK, H), lambda i: (0, i, 0)),
        pl.BlockSpec((1, BLK, H), lambda i: (1, i, 0)),
        pl.BlockSpec((H, H), lambda i: (0, 0)),
        pl.BlockSpec((1, H), lambda i: (0, 0)),
    ],
    out_specs=pl.BlockSpec((BLK, H), lambda i: (i, 0)),
    out_shape=jax.ShapeDtypeStruct((N, H), jnp.float32),
)

_tc_out = pl.pallas_call(
    _tc_out_body,
    grid=(GRID,),
    in_specs=[
        pl.BlockSpec((1, BLK, H), lambda i: (0, i, 0)),
        pl.BlockSpec((1, BLK, H), lambda i: (1, i, 0)),
        pl.BlockSpec((1, BLK, H), lambda i: (0, i, 0)),
        pl.BlockSpec((1, BLK, H), lambda i: (1, i, 0)),
        pl.BlockSpec((H, H), lambda i: (0, 0)),
        pl.BlockSpec((1, H), lambda i: (0, 0)),
        pl.BlockSpec((1, H), lambda i: (0, 0)),
        pl.BlockSpec((1, H), lambda i: (0, 0)),
    ],
    out_specs=pl.BlockSpec((1, H), lambda i: (0, 0)),
    out_shape=jax.ShapeDtypeStruct((1, H), jnp.float32),
    scratch_shapes=[pltpu.VMEM((1, H), jnp.float32)],
)


def kernel(x, edge_index, W_in, b_in, W1, b1, W2, b2, W_out, b_out):
    src = edge_index[0].astype(jnp.int32).reshape(NW * NCHUNK, 1, CH)
    dst = edge_index[1].astype(jnp.int32).reshape(NW * NCHUNK, 1, CH)
    zeros = jnp.zeros((N, H), jnp.float32)
    ones = jnp.ones((DCH, H), jnp.float32)

    deg = _sc_deg(dst, zeros, ones)
    h0 = _tc_in(x, W_in, b_in.reshape(1, H))
    agg1 = _sc_agg(h0, src, dst, zeros)
    h1 = _tc_mid(agg1, agg1, deg, deg, W1, b1.reshape(1, H))
    agg2 = _sc_agg(h1, src, dst, zeros)
    y = _tc_out(agg2, agg2, deg, deg, W2, b2.reshape(1, H),
                W_out.reshape(1, H),
                jnp.broadcast_to(b_out.reshape(1, 1), (1, H)))
    return y[0, 0]
